# baseline (speedup 1.0000x reference)
import functools

import jax
import jax.numpy as jnp
from jax import lax
from jax.experimental import pallas as pl
from jax.experimental.pallas import tpu as pltpu
from jax.experimental.pallas import tpu_sc as plsc

_NC = 2
_NS = 16
_NW = _NC * _NS


def _gather_kernel(n_tokens, h32, chunk):
    per_w = n_tokens // _NW
    n_chunks = per_w // chunk
    mesh = plsc.VectorSubcoreMesh(core_axis_name="c", subcore_axis_name="s")

    @functools.partial(
        pl.kernel,
        mesh=mesh,
        out_type=jax.ShapeDtypeStruct((n_tokens, h32), jnp.int32),
        scratch_types=[
            pltpu.VMEM((per_w,), jnp.int32),
            pltpu.VMEM((chunk, h32), jnp.int32),
            pltpu.SemaphoreType.DMA,
        ],
    )
    def body(idx_hbm, w_hbm, out_hbm, idx_v, rows_v, sem):
        wid = lax.axis_index("s") * _NC + lax.axis_index("c")
        base = wid * per_w
        pltpu.sync_copy(idx_hbm.at[pl.ds(base, per_w)], idx_v)

        def step(g, carry):
            pltpu.async_copy(
                w_hbm.at[idx_v.at[pl.ds(g * chunk, chunk)]], rows_v, sem
            ).wait()
            pltpu.sync_copy(rows_v, out_hbm.at[pl.ds(base + g * chunk, chunk)])
            return carry

        lax.fori_loop(0, n_chunks, step, 0)

    return body


def kernel(input_ids, W):
    b, s = input_ids.shape
    vocab, hidden = W.shape
    n = b * s
    h32 = hidden // 2
    idx = input_ids.reshape(n)
    w32 = lax.bitcast_convert_type(W.reshape(vocab, h32, 2), jnp.int32)
    out32 = _gather_kernel(n, h32, chunk=16)(idx, w32)
    out = lax.bitcast_convert_type(out32, jnp.float16)
    return out.reshape(b, s, hidden)


# speedup vs baseline: 82.6653x; 82.6653x over previous
import functools

import jax
import jax.numpy as jnp
from jax import lax
from jax.experimental import pallas as pl
from jax.experimental.pallas import tpu as pltpu
from jax.experimental.pallas import tpu_sc as plsc

_NC = 2
_NS = 16
_NW = _NC * _NS
_TB = 16


def _gather_kernel(b_sz, s_sz, hidden):
    wpb = _NW // b_sz
    per_w = s_sz // wpb
    n_blk = per_w // _TB
    h2 = hidden // 4
    mesh = plsc.VectorSubcoreMesh(core_axis_name="c", subcore_axis_name="s")

    @functools.partial(
        pl.kernel,
        mesh=mesh,
        out_type=jax.ShapeDtypeStruct((b_sz, s_sz, hidden), jnp.float16),
        scratch_types=[
            pltpu.VMEM((per_w,), jnp.int32),
            pltpu.VMEM((2, _TB, h2), jnp.int32),
            pltpu.VMEM((_TB // 2, h2), jnp.int32),
            pltpu.SemaphoreType.DMA,
            pltpu.SemaphoreType.DMA,
        ],
    )
    def body(idx_hbm, w_hbm, out_hbm, idx_v, g_v, ob_v, sem_g, sem_o):
        wid = lax.axis_index("s") * _NC + lax.axis_index("c")
        b = wid // wpb
        woff = pl.multiple_of((wid % wpb) * per_w, per_w)
        w32 = w_hbm.bitcast(jnp.int32)
        out32 = out_hbm.bitcast(jnp.int32)
        pltpu.sync_copy(idx_hbm.at[b, pl.ds(woff, per_w)], idx_v)

        def start_gather(k):
            blk, half = divmod(k, 2)
            avec = lax.shift_right_logical(idx_v[pl.ds(_TB * blk, _TB)], 1)
            return pltpu.async_copy(
                w32.at[avec, pl.ds(half * h2, h2)], g_v.at[k % 2], sem_g
            )

        def extract(k):
            blk = k // 2
            tvec = idx_v[pl.ds(_TB * blk, _TB)]
            shs = [(tvec[i] & 1) << 4 for i in range(_TB)]

            def cg_body(cg, carry):
                base = cg * 16
                for u in range(_TB // 2):
                    v0 = g_v[k % 2, 2 * u, pl.ds(base, 16)]
                    v1 = g_v[k % 2, 2 * u + 1, pl.ds(base, 16)]
                    lo = (v0 >> shs[2 * u]) & 0xFFFF
                    w = lo | ((v1 >> shs[2 * u + 1]) << 16)
                    ob_v[u, pl.ds(base, 16)] = w
                return carry

            lax.fori_loop(0, h2 // 16, cg_body, 0)

        n_g = 2 * n_blk
        handles = [None] * n_g
        handles[0] = start_gather(0)
        flush = None
        for k in range(n_g):
            if k + 1 < n_g:
                handles[k + 1] = start_gather(k + 1)
            handles[k].wait()
            if flush is not None:
                flush.wait()
            extract(k)
            blk, half = divmod(k, 2)
            flush = pltpu.async_copy(
                ob_v,
                out32.at[
                    b,
                    pl.ds(
                        pl.multiple_of(
                            woff // 2 + (_TB // 2) * blk, _TB // 2
                        ),
                        _TB // 2,
                    ),
                    pl.ds(half * h2, h2),
                ],
                sem_o,
            )
        flush.wait()

    return body


def kernel(input_ids, W):
    b, s = input_ids.shape
    vocab, hidden = W.shape
    return _gather_kernel(b, s, hidden)(input_ids, W)
